# baseline (speedup 1.0000x reference)
import functools

import jax
import jax.numpy as jnp
from jax import lax
from jax.experimental import pallas as pl
from jax.experimental.pallas import tpu as pltpu
from jax.experimental.pallas import tpu_sc as plsc

N = 10000
NPAD = 10240
E = 320000
D_IN, D_H, D_OUT = 128, 128, 64

NC, NS = 2, 16
NW = NC * NS
EPAD = 327680
EPT = EPAD // NW
CH = 80
NCHUNK = EPT // CH
RPT = NPAD // NS

_mesh = plsc.VectorSubcoreMesh(core_axis_name="c", subcore_axis_name="s")
_sc_params = pltpu.CompilerParams(needs_layout_passes=False,
                                  use_tc_tiling_on_sc=False)


def _deg_body(dst_hbm, w_hbm, deg0_hbm, deg1_hbm, part_hbm,
              degv, idxb, wb, accb, tmp):
    cid = lax.axis_index("c")
    sid = lax.axis_index("s")
    wid = cid * NS + sid

    def zero(i, _):
        degv[pl.ds(i * 16, 16)] = jnp.zeros((16,), jnp.float32)
        return 0
    lax.fori_loop(0, NPAD // 16, zero, 0)

    ebase = wid * EPT
    pltpu.sync_copy(dst_hbm.at[pl.ds(ebase, EPT)], idxb)
    pltpu.sync_copy(w_hbm.at[pl.ds(ebase, EPT)], wb)

    def inner(i, _):
        sl = pl.ds(i * 16, 16)
        plsc.addupdate_scatter(degv, [idxb[sl]], wb[sl])
        return 0
    lax.fori_loop(0, EPT // 16, inner, 0)

    pltpu.sync_copy(degv, part_hbm.at[cid, sid])
    plsc.subcore_barrier()
    rbase = sid * RPT
    pltpu.sync_copy(part_hbm.at[cid, 0, pl.ds(rbase, RPT)], accb)

    def red(t, _):
        pltpu.sync_copy(part_hbm.at[cid, t, pl.ds(rbase, RPT)], tmp)

        def addv(i, _):
            sl = pl.ds(i * 16, 16)
            accb[sl] = accb[sl] + tmp[sl]
            return 0
        lax.fori_loop(0, RPT // 16, addv, 0)
        return 0
    lax.fori_loop(1, NS, red, 0)

    @pl.when(cid == 0)
    def _():
        pltpu.sync_copy(accb, deg0_hbm.at[pl.ds(rbase, RPT)])

    @pl.when(cid == 1)
    def _():
        pltpu.sync_copy(accb, deg1_hbm.at[pl.ds(rbase, RPT)])


_deg_call = functools.partial(
    pl.kernel,
    out_type=(jax.ShapeDtypeStruct((NPAD,), jnp.float32),
              jax.ShapeDtypeStruct((NPAD,), jnp.float32),
              jax.ShapeDtypeStruct((NC, NS, NPAD), jnp.float32)),
    mesh=_mesh,
    scratch_types=[
        pltpu.VMEM((NPAD,), jnp.float32),
        pltpu.VMEM((EPT,), jnp.int32),
        pltpu.VMEM((EPT,), jnp.float32),
        pltpu.VMEM((RPT,), jnp.float32),
        pltpu.VMEM((RPT,), jnp.float32),
    ],
    compiler_params=_sc_params,
)(_deg_body)


NBUF = 4
NQ = NCHUNK // NBUF


def _agg_body(y_hbm, src_hbm, dst_hbm, w_hbm, z_hbm, out_hbm,
              acc, sb, db, wb, *bufs_and_sems, d):
    rows = bufs_and_sems[:NBUF]
    gsem = bufs_and_sems[NBUF:2 * NBUF]
    ssem = bufs_and_sems[2 * NBUF:3 * NBUF]
    ps, pd, pw = bufs_and_sems[3 * NBUF:3 * NBUF + 3]
    cid = lax.axis_index("c")
    sid = lax.axis_index("s")
    wid = cid * NS + sid
    rbase = sid * RPT

    pltpu.sync_copy(src_hbm.at[wid, 0], sb.at[0])
    pltpu.sync_copy(dst_hbm.at[wid, 0], db.at[0])
    pltpu.sync_copy(w_hbm.at[wid, 0], wb.at[0])
    pltpu.async_copy(src_hbm.at[wid, 1], sb.at[1], ps)
    pltpu.async_copy(dst_hbm.at[wid, 1], db.at[1], pd)
    pltpu.async_copy(w_hbm.at[wid, 1], wb.at[1], pw)

    for j in range(NBUF):
        pltpu.async_copy(y_hbm.at[sb.at[0, j]], rows[j], gsem[j])

    pltpu.sync_copy(z_hbm, acc.at[pl.ds(rbase, RPT)])
    plsc.subcore_barrier()

    def round_(q, _):
        p = lax.rem(q, 2)
        pn = 1 - p
        for j in range(NBUF):
            pltpu.make_async_copy(y_hbm.at[sb.at[p, j]], rows[j],
                                  gsem[j]).wait()

            def scale(g, _):
                w16 = wb[p, j, pl.ds(g * 16, 16)]
                for e in range(16):
                    we = w16[e]
                    r = g * 16 + e
                    for jv in range(d // 16):
                        sl = pl.ds(jv * 16, 16)
                        rows[j][r, sl] = rows[j][r, sl] * we
                return 0
            lax.fori_loop(0, CH // 16, scale, 0)

            pltpu.async_copy(rows[j], acc.at[db.at[p, j]], ssem[j], add=True)
        for j in range(NBUF):
            pltpu.make_async_copy(rows[j], acc.at[db.at[p, j]],
                                  ssem[j]).wait()

        @pl.when(q < NQ - 1)
        def _():
            pltpu.make_async_copy(src_hbm.at[wid, q + 1], sb.at[pn],
                                  ps).wait()
            pltpu.make_async_copy(dst_hbm.at[wid, q + 1], db.at[pn],
                                  pd).wait()
            pltpu.make_async_copy(w_hbm.at[wid, q + 1], wb.at[pn],
                                  pw).wait()

            @pl.when(q < NQ - 2)
            def _():
                pltpu.async_copy(src_hbm.at[wid, q + 2], sb.at[p], ps)
                pltpu.async_copy(dst_hbm.at[wid, q + 2], db.at[p], pd)
                pltpu.async_copy(w_hbm.at[wid, q + 2], wb.at[p], pw)

            for j in range(NBUF):
                pltpu.async_copy(y_hbm.at[sb.at[pn, j]], rows[j], gsem[j])
        return 0
    lax.fori_loop(0, NQ, round_, 0)

    plsc.subcore_barrier()
    pltpu.sync_copy(acc.at[pl.ds(rbase, RPT)],
                    out_hbm.at[cid, pl.ds(rbase, RPT)])


def _make_agg(d):
    return pl.kernel(
        functools.partial(_agg_body, d=d),
        out_type=jax.ShapeDtypeStruct((NC, NPAD, d), jnp.float32),
        mesh=_mesh,
        scratch_types=[
            pltpu.VMEM_SHARED((NPAD, d), jnp.float32),
            pltpu.VMEM((2, NBUF, CH), jnp.int32),
            pltpu.VMEM((2, NBUF, CH), jnp.int32),
            pltpu.VMEM((2, NBUF, CH), jnp.float32),
            *[pltpu.VMEM((CH, d), jnp.float32) for _ in range(NBUF)],
            *[pltpu.SemaphoreType.DMA for _ in range(2 * NBUF + 3)],
        ],
        compiler_params=_sc_params,
    )


_agg_h = _make_agg(D_H)
_agg_o = _make_agg(D_OUT)


BM = 1024
GRID = NPAD // BM


def _tc1_body(x_ref, w_ref, d0_ref, d1_ref, y_ref, dinv_ref):
    deg = d0_ref[...] + d1_ref[...] + 1.0
    dinv = lax.rsqrt(deg)
    xw = jnp.dot(x_ref[...], w_ref[...], preferred_element_type=jnp.float32)
    y_ref[...] = xw * dinv
    dinv_ref[...] = dinv


def _tc1(x, W1, d0, d1):
    return pl.pallas_call(
        _tc1_body,
        grid=(GRID,),
        in_specs=[
            pl.BlockSpec((BM, D_IN), lambda i: (i, 0)),
            pl.BlockSpec((D_IN, D_H), lambda i: (0, 0)),
            pl.BlockSpec((BM, 1), lambda i: (i, 0)),
            pl.BlockSpec((BM, 1), lambda i: (i, 0)),
        ],
        out_specs=[
            pl.BlockSpec((BM, D_H), lambda i: (i, 0)),
            pl.BlockSpec((BM, 1), lambda i: (i, 0)),
        ],
        out_shape=[
            jax.ShapeDtypeStruct((N, D_H), jnp.float32),
            jax.ShapeDtypeStruct((NPAD, 1), jnp.float32),
        ],
    )(x, W1, d0, d1)


def _tc2_body(p_ref, y1_ref, dinv_ref, b1_ref, w2_ref, y2_ref):
    agg = p_ref[0] + p_ref[1] + y1_ref[...]
    h = jnp.maximum(dinv_ref[...] * agg + b1_ref[...], 0.0)
    y2_ref[...] = jnp.dot(h, w2_ref[...],
                          preferred_element_type=jnp.float32) * dinv_ref[...]


def _tc2(p1, y1, dinv, b1, W2):
    return pl.pallas_call(
        _tc2_body,
        grid=(GRID,),
        in_specs=[
            pl.BlockSpec((NC, BM, D_H), lambda i: (0, i, 0)),
            pl.BlockSpec((BM, D_H), lambda i: (i, 0)),
            pl.BlockSpec((BM, 1), lambda i: (i, 0)),
            pl.BlockSpec((1, D_H), lambda i: (0, 0)),
            pl.BlockSpec((D_H, D_OUT), lambda i: (0, 0)),
        ],
        out_specs=pl.BlockSpec((BM, D_OUT), lambda i: (i, 0)),
        out_shape=jax.ShapeDtypeStruct((N, D_OUT), jnp.float32),
    )(p1, y1, dinv, b1, W2)


def _tc3_body(p_ref, y2_ref, dinv_ref, b2_ref, o_ref):
    agg = p_ref[0] + p_ref[1] + y2_ref[...]
    o = dinv_ref[...] * agg + b2_ref[...]
    m = jnp.max(o, axis=-1, keepdims=True)
    e = jnp.exp(o - m)
    lse = jnp.log(jnp.sum(e, axis=-1, keepdims=True))
    o_ref[...] = (o - m) - lse


def _tc3(p2, y2, dinv, b2):
    return pl.pallas_call(
        _tc3_body,
        grid=(GRID,),
        in_specs=[
            pl.BlockSpec((NC, BM, D_OUT), lambda i: (0, i, 0)),
            pl.BlockSpec((BM, D_OUT), lambda i: (i, 0)),
            pl.BlockSpec((BM, 1), lambda i: (i, 0)),
            pl.BlockSpec((1, D_OUT), lambda i: (0, 0)),
        ],
        out_specs=pl.BlockSpec((BM, D_OUT), lambda i: (i, 0)),
        out_shape=jax.ShapeDtypeStruct((N, D_OUT), jnp.float32),
    )(p2, y2, dinv, b2)


def kernel(x, edge_index, edge_weight, W1, b1, W2, b2):
    src = edge_index[0].astype(jnp.int32)
    dst = edge_index[1].astype(jnp.int32)
    ew = edge_weight.astype(jnp.float32)
    npad_e = EPAD - E
    src = jnp.concatenate([src, jnp.zeros((npad_e,), jnp.int32)])
    dst = jnp.concatenate([dst, jnp.zeros((npad_e,), jnp.int32)])
    ew = jnp.concatenate([ew, jnp.zeros((npad_e,), jnp.float32)])

    deg0, deg1, _ = _deg_call(dst, ew)
    d0 = deg0.reshape(NPAD, 1)
    d1 = deg1.reshape(NPAD, 1)

    y1, dinv = _tc1(x, W1, d0, d1)

    src3 = src.reshape(NW, NQ, NBUF, CH)
    dst3 = dst.reshape(NW, NQ, NBUF, CH)
    ew3 = ew.reshape(NW, NQ, NBUF, CH)

    z_h = jnp.zeros((RPT, D_H), jnp.float32)
    p1 = _agg_h(y1, src3, dst3, ew3, z_h)

    y2 = _tc2(p1, y1, dinv, b1.reshape(1, D_H), W2)

    z_o = jnp.zeros((RPT, D_OUT), jnp.float32)
    p2 = _agg_o(y2, src3, dst3, ew3, z_o)

    return _tc3(p2, y2, dinv, b2.reshape(1, D_OUT))


# speedup vs baseline: 1.0038x; 1.0038x over previous
import functools

import jax
import jax.numpy as jnp
from jax import lax
from jax.experimental import pallas as pl
from jax.experimental.pallas import tpu as pltpu
from jax.experimental.pallas import tpu_sc as plsc

N = 10000
NPAD = 10240
E = 320000
D_IN, D_H, D_OUT = 128, 128, 64

NC, NS = 2, 16
NW = NC * NS
EPAD = 327680
EPT = EPAD // NW
CH = 80
NCHUNK = EPT // CH
RPT = NPAD // NS

_mesh = plsc.VectorSubcoreMesh(core_axis_name="c", subcore_axis_name="s")
_sc_params = pltpu.CompilerParams(needs_layout_passes=False,
                                  use_tc_tiling_on_sc=False)


def _deg_body(dst_hbm, w_hbm, deg0_hbm, deg1_hbm, part_hbm,
              degv, idxb, wb, accb, tmp):
    cid = lax.axis_index("c")
    sid = lax.axis_index("s")
    wid = cid * NS + sid

    def zero(i, _):
        degv[pl.ds(i * 16, 16)] = jnp.zeros((16,), jnp.float32)
        return 0
    lax.fori_loop(0, NPAD // 16, zero, 0)

    ebase = wid * EPT
    pltpu.sync_copy(dst_hbm.at[pl.ds(ebase, EPT)], idxb)
    pltpu.sync_copy(w_hbm.at[pl.ds(ebase, EPT)], wb)

    def inner(i, _):
        sl = pl.ds(i * 16, 16)
        plsc.addupdate_scatter(degv, [idxb[sl]], wb[sl])
        return 0
    lax.fori_loop(0, EPT // 16, inner, 0)

    pltpu.sync_copy(degv, part_hbm.at[cid, sid])
    plsc.subcore_barrier()
    rbase = sid * RPT
    pltpu.sync_copy(part_hbm.at[cid, 0, pl.ds(rbase, RPT)], accb)

    def red(t, _):
        pltpu.sync_copy(part_hbm.at[cid, t, pl.ds(rbase, RPT)], tmp)

        def addv(i, _):
            sl = pl.ds(i * 16, 16)
            accb[sl] = accb[sl] + tmp[sl]
            return 0
        lax.fori_loop(0, RPT // 16, addv, 0)
        return 0
    lax.fori_loop(1, NS, red, 0)

    @pl.when(cid == 0)
    def _():
        pltpu.sync_copy(accb, deg0_hbm.at[pl.ds(rbase, RPT)])

    @pl.when(cid == 1)
    def _():
        pltpu.sync_copy(accb, deg1_hbm.at[pl.ds(rbase, RPT)])


_deg_call = functools.partial(
    pl.kernel,
    out_type=(jax.ShapeDtypeStruct((NPAD,), jnp.float32),
              jax.ShapeDtypeStruct((NPAD,), jnp.float32),
              jax.ShapeDtypeStruct((NC, NS, NPAD), jnp.float32)),
    mesh=_mesh,
    scratch_types=[
        pltpu.VMEM((NPAD,), jnp.float32),
        pltpu.VMEM((EPT,), jnp.int32),
        pltpu.VMEM((EPT,), jnp.float32),
        pltpu.VMEM((RPT,), jnp.float32),
        pltpu.VMEM((RPT,), jnp.float32),
    ],
    compiler_params=_sc_params,
)(_deg_body)


NBUF = 4
NQ = NCHUNK // NBUF


def _agg_body(y_hbm, src_hbm, dst_hbm, w_hbm, z_hbm, out_hbm,
              acc, sb, db, wb, *bufs_and_sems, d):
    rows = bufs_and_sems[:NBUF]
    gsem = bufs_and_sems[NBUF:2 * NBUF]
    ssem = bufs_and_sems[2 * NBUF:3 * NBUF]
    ps, pd, pw = bufs_and_sems[3 * NBUF:3 * NBUF + 3]
    cid = lax.axis_index("c")
    sid = lax.axis_index("s")
    wid = cid * NS + sid
    rbase = sid * RPT

    pltpu.sync_copy(src_hbm.at[wid, 0], sb.at[0])
    pltpu.sync_copy(dst_hbm.at[wid, 0], db.at[0])
    pltpu.sync_copy(w_hbm.at[wid, 0], wb.at[0])
    pltpu.async_copy(src_hbm.at[wid, 1], sb.at[1], ps)
    pltpu.async_copy(dst_hbm.at[wid, 1], db.at[1], pd)
    pltpu.async_copy(w_hbm.at[wid, 1], wb.at[1], pw)

    for j in range(NBUF):
        pltpu.async_copy(y_hbm.at[sb.at[0, j]], rows[j], gsem[j])

    pltpu.sync_copy(z_hbm, acc.at[pl.ds(rbase, RPT)])
    plsc.subcore_barrier()

    def round_(q, _):
        p = lax.rem(q, 2)
        pn = 1 - p
        for j in range(NBUF):
            with jax.named_scope("gwait"):
                pltpu.make_async_copy(y_hbm.at[sb.at[p, j]], rows[j],
                                      gsem[j]).wait()

            def scale(g, _):
                w16 = wb[p, j, pl.ds(g * 16, 16)]
                for e in range(16):
                    we = w16[e]
                    r = g * 16 + e
                    for jv in range(d // 16):
                        sl = pl.ds(jv * 16, 16)
                        rows[j][r, sl] = rows[j][r, sl] * we
                return 0
            with jax.named_scope("scale"):
                lax.fori_loop(0, CH // 16, scale, 0)

            pltpu.async_copy(rows[j], acc.at[db.at[p, j]], ssem[j], add=True)
        with jax.named_scope("swait"):
            for j in range(NBUF):
                pltpu.make_async_copy(rows[j], acc.at[db.at[p, j]],
                                      ssem[j]).wait()

        @pl.when(q < NQ - 1)
        def _():
            pltpu.make_async_copy(src_hbm.at[wid, q + 1], sb.at[pn],
                                  ps).wait()
            pltpu.make_async_copy(dst_hbm.at[wid, q + 1], db.at[pn],
                                  pd).wait()
            pltpu.make_async_copy(w_hbm.at[wid, q + 1], wb.at[pn],
                                  pw).wait()

            @pl.when(q < NQ - 2)
            def _():
                pltpu.async_copy(src_hbm.at[wid, q + 2], sb.at[p], ps)
                pltpu.async_copy(dst_hbm.at[wid, q + 2], db.at[p], pd)
                pltpu.async_copy(w_hbm.at[wid, q + 2], wb.at[p], pw)

            for j in range(NBUF):
                pltpu.async_copy(y_hbm.at[sb.at[pn, j]], rows[j], gsem[j])
        return 0
    lax.fori_loop(0, NQ, round_, 0)

    plsc.subcore_barrier()
    pltpu.sync_copy(acc.at[pl.ds(rbase, RPT)],
                    out_hbm.at[cid, pl.ds(rbase, RPT)])


def _make_agg(d):
    return pl.kernel(
        functools.partial(_agg_body, d=d),
        out_type=jax.ShapeDtypeStruct((NC, NPAD, d), jnp.float32),
        mesh=_mesh,
        scratch_types=[
            pltpu.VMEM_SHARED((NPAD, d), jnp.float32),
            pltpu.VMEM((2, NBUF, CH), jnp.int32),
            pltpu.VMEM((2, NBUF, CH), jnp.int32),
            pltpu.VMEM((2, NBUF, CH), jnp.float32),
            *[pltpu.VMEM((CH, d), jnp.float32) for _ in range(NBUF)],
            *[pltpu.SemaphoreType.DMA for _ in range(2 * NBUF + 3)],
        ],
        compiler_params=_sc_params,
    )


_agg_h = _make_agg(D_H)
_agg_o = _make_agg(D_OUT)


BM = 1024
GRID = NPAD // BM


def _tc1_body(x_ref, w_ref, d0_ref, d1_ref, y_ref, dinv_ref):
    deg = d0_ref[...] + d1_ref[...] + 1.0
    dinv = lax.rsqrt(deg)
    xw = jnp.dot(x_ref[...], w_ref[...], preferred_element_type=jnp.float32)
    y_ref[...] = xw * dinv
    dinv_ref[...] = dinv


def _tc1(x, W1, d0, d1):
    return pl.pallas_call(
        _tc1_body,
        grid=(GRID,),
        in_specs=[
            pl.BlockSpec((BM, D_IN), lambda i: (i, 0)),
            pl.BlockSpec((D_IN, D_H), lambda i: (0, 0)),
            pl.BlockSpec((BM, 1), lambda i: (i, 0)),
            pl.BlockSpec((BM, 1), lambda i: (i, 0)),
        ],
        out_specs=[
            pl.BlockSpec((BM, D_H), lambda i: (i, 0)),
            pl.BlockSpec((BM, 1), lambda i: (i, 0)),
        ],
        out_shape=[
            jax.ShapeDtypeStruct((N, D_H), jnp.float32),
            jax.ShapeDtypeStruct((NPAD, 1), jnp.float32),
        ],
    )(x, W1, d0, d1)


def _tc2_body(p_ref, y1_ref, dinv_ref, b1_ref, w2_ref, y2_ref):
    agg = p_ref[0] + p_ref[1] + y1_ref[...]
    h = jnp.maximum(dinv_ref[...] * agg + b1_ref[...], 0.0)
    y2_ref[...] = jnp.dot(h, w2_ref[...],
                          preferred_element_type=jnp.float32) * dinv_ref[...]


def _tc2(p1, y1, dinv, b1, W2):
    return pl.pallas_call(
        _tc2_body,
        grid=(GRID,),
        in_specs=[
            pl.BlockSpec((NC, BM, D_H), lambda i: (0, i, 0)),
            pl.BlockSpec((BM, D_H), lambda i: (i, 0)),
            pl.BlockSpec((BM, 1), lambda i: (i, 0)),
            pl.BlockSpec((1, D_H), lambda i: (0, 0)),
            pl.BlockSpec((D_H, D_OUT), lambda i: (0, 0)),
        ],
        out_specs=pl.BlockSpec((BM, D_OUT), lambda i: (i, 0)),
        out_shape=jax.ShapeDtypeStruct((N, D_OUT), jnp.float32),
    )(p1, y1, dinv, b1, W2)


def _tc3_body(p_ref, y2_ref, dinv_ref, b2_ref, o_ref):
    agg = p_ref[0] + p_ref[1] + y2_ref[...]
    o = dinv_ref[...] * agg + b2_ref[...]
    m = jnp.max(o, axis=-1, keepdims=True)
    e = jnp.exp(o - m)
    lse = jnp.log(jnp.sum(e, axis=-1, keepdims=True))
    o_ref[...] = (o - m) - lse


def _tc3(p2, y2, dinv, b2):
    return pl.pallas_call(
        _tc3_body,
        grid=(GRID,),
        in_specs=[
            pl.BlockSpec((NC, BM, D_OUT), lambda i: (0, i, 0)),
            pl.BlockSpec((BM, D_OUT), lambda i: (i, 0)),
            pl.BlockSpec((BM, 1), lambda i: (i, 0)),
            pl.BlockSpec((1, D_OUT), lambda i: (0, 0)),
        ],
        out_specs=pl.BlockSpec((BM, D_OUT), lambda i: (i, 0)),
        out_shape=jax.ShapeDtypeStruct((N, D_OUT), jnp.float32),
    )(p2, y2, dinv, b2)


def kernel(x, edge_index, edge_weight, W1, b1, W2, b2):
    src = edge_index[0].astype(jnp.int32)
    dst = edge_index[1].astype(jnp.int32)
    ew = edge_weight.astype(jnp.float32)
    npad_e = EPAD - E
    src = jnp.concatenate([src, jnp.zeros((npad_e,), jnp.int32)])
    dst = jnp.concatenate([dst, jnp.zeros((npad_e,), jnp.int32)])
    ew = jnp.concatenate([ew, jnp.zeros((npad_e,), jnp.float32)])

    deg0, deg1, _ = _deg_call(dst, ew)
    d0 = deg0.reshape(NPAD, 1)
    d1 = deg1.reshape(NPAD, 1)

    y1, dinv = _tc1(x, W1, d0, d1)

    src3 = src.reshape(NW, NQ, NBUF, CH)
    dst3 = dst.reshape(NW, NQ, NBUF, CH)
    ew3 = ew.reshape(NW, NQ, NBUF, CH)

    z_h = jnp.zeros((RPT, D_H), jnp.float32)
    p1 = _agg_h(y1, src3, dst3, ew3, z_h)

    y2 = _tc2(p1, y1, dinv, b1.reshape(1, D_H), W2)

    z_o = jnp.zeros((RPT, D_OUT), jnp.float32)
    p2 = _agg_o(y2, src3, dst3, ew3, z_o)

    return _tc3(p2, y2, dinv, b2.reshape(1, D_OUT))


# speedup vs baseline: 1.2824x; 1.2775x over previous
import functools

import jax
import jax.numpy as jnp
from jax import lax
from jax.experimental import pallas as pl
from jax.experimental.pallas import tpu as pltpu
from jax.experimental.pallas import tpu_sc as plsc

N = 10000
NPAD = 10240
E = 320000
D_IN, D_H, D_OUT = 128, 128, 64

NC, NS = 2, 16
NW = NC * NS
EPAD = 327680
EPT = EPAD // NW
CH = 80
NCHUNK = EPT // CH
RPT = NPAD // NS

_mesh = plsc.VectorSubcoreMesh(core_axis_name="c", subcore_axis_name="s")
_sc_params = pltpu.CompilerParams(needs_layout_passes=False,
                                  use_tc_tiling_on_sc=False)


def _deg_body(dst_hbm, w_hbm, deg0_hbm, deg1_hbm, part_hbm,
              degv, idxb, wb, accb, tmp):
    cid = lax.axis_index("c")
    sid = lax.axis_index("s")
    wid = cid * NS + sid

    def zero(i, _):
        degv[pl.ds(i * 16, 16)] = jnp.zeros((16,), jnp.float32)
        return 0
    lax.fori_loop(0, NPAD // 16, zero, 0)

    ebase = wid * EPT
    pltpu.sync_copy(dst_hbm.at[pl.ds(ebase, EPT)], idxb)
    pltpu.sync_copy(w_hbm.at[pl.ds(ebase, EPT)], wb)

    def inner(i, _):
        sl = pl.ds(i * 16, 16)
        plsc.addupdate_scatter(degv, [idxb[sl]], wb[sl])
        return 0
    lax.fori_loop(0, EPT // 16, inner, 0)

    pltpu.sync_copy(degv, part_hbm.at[cid, sid])
    plsc.subcore_barrier()
    rbase = sid * RPT
    pltpu.sync_copy(part_hbm.at[cid, 0, pl.ds(rbase, RPT)], accb)

    def red(t, _):
        pltpu.sync_copy(part_hbm.at[cid, t, pl.ds(rbase, RPT)], tmp)

        def addv(i, _):
            sl = pl.ds(i * 16, 16)
            accb[sl] = accb[sl] + tmp[sl]
            return 0
        lax.fori_loop(0, RPT // 16, addv, 0)
        return 0
    lax.fori_loop(1, NS, red, 0)

    @pl.when(cid == 0)
    def _():
        pltpu.sync_copy(accb, deg0_hbm.at[pl.ds(rbase, RPT)])

    @pl.when(cid == 1)
    def _():
        pltpu.sync_copy(accb, deg1_hbm.at[pl.ds(rbase, RPT)])


_deg_call = functools.partial(
    pl.kernel,
    out_type=(jax.ShapeDtypeStruct((NPAD,), jnp.float32),
              jax.ShapeDtypeStruct((NPAD,), jnp.float32),
              jax.ShapeDtypeStruct((NC, NS, NPAD), jnp.float32)),
    mesh=_mesh,
    scratch_types=[
        pltpu.VMEM((NPAD,), jnp.float32),
        pltpu.VMEM((EPT,), jnp.int32),
        pltpu.VMEM((EPT,), jnp.float32),
        pltpu.VMEM((RPT,), jnp.float32),
        pltpu.VMEM((RPT,), jnp.float32),
    ],
    compiler_params=_sc_params,
)(_deg_body)


NBUF = 4
NQ = NCHUNK // NBUF


NCH_TOT = EPAD // CH


def _agg_body(y_hbm, src_hbm, dst_hbm, w_hbm, out_hbm,
              acc, sb, db, wb, *bufs_and_sems, d, n0):
    rows = bufs_and_sems[:NBUF]
    gsem = bufs_and_sems[NBUF:2 * NBUF]
    ssem = bufs_and_sems[2 * NBUF:3 * NBUF]
    ps, pd, pw = bufs_and_sems[3 * NBUF:3 * NBUF + 3]
    n1 = (NCH_TOT - NS * n0) // NS
    cid = lax.axis_index("c")
    sid = lax.axis_index("s")
    rbase = sid * RPT
    nq = jnp.where(cid == 0, n0 // NBUF, n1 // NBUF)
    cbase = jnp.where(cid == 0, sid * n0, NS * n0 + sid * n1)

    pltpu.sync_copy(src_hbm.at[pl.ds(cbase, NBUF)], sb.at[0])
    pltpu.sync_copy(dst_hbm.at[pl.ds(cbase, NBUF)], db.at[0])
    pltpu.sync_copy(w_hbm.at[pl.ds(cbase, NBUF)], wb.at[0])
    pltpu.async_copy(src_hbm.at[pl.ds(cbase + NBUF, NBUF)], sb.at[1], ps)
    pltpu.async_copy(dst_hbm.at[pl.ds(cbase + NBUF, NBUF)], db.at[1], pd)
    pltpu.async_copy(w_hbm.at[pl.ds(cbase + NBUF, NBUF)], wb.at[1], pw)

    def zrow(r, _):
        for jv in range(d // 16):
            rows[0][r, pl.ds(jv * 16, 16)] = jnp.zeros((16,), jnp.float32)
        return 0
    lax.fori_loop(0, CH, zrow, 0)
    for t in range(RPT // CH):
        pltpu.sync_copy(rows[0], acc.at[pl.ds(rbase + t * CH, CH)])

    for j in range(NBUF):
        pltpu.async_copy(y_hbm.at[sb.at[0, j]], rows[j], gsem[j])

    plsc.subcore_barrier()

    def round_(q, _):
        p = lax.rem(q, 2)
        pn = 1 - p
        for j in range(NBUF):
            with jax.named_scope("gwait"):
                pltpu.make_async_copy(y_hbm.at[sb.at[p, j]], rows[j],
                                      gsem[j]).wait()

            with jax.named_scope("scale"):
                @plsc.parallel_loop(0, CH // 16, unroll=2)
                def scale(g):
                    w16 = wb[p, j, pl.ds(g * 16, 16)]
                    for e in range(16):
                        we = w16[e]
                        r = g * 16 + e
                        for jv in range(d // 16):
                            sl = pl.ds(jv * 16, 16)
                            rows[j][r, sl] = rows[j][r, sl] * we

            pltpu.async_copy(rows[j], acc.at[db.at[p, j]], ssem[j], add=True)
        with jax.named_scope("swait"):
            for j in range(NBUF):
                pltpu.make_async_copy(rows[j], acc.at[db.at[p, j]],
                                      ssem[j]).wait()

        @pl.when(q < nq - 1)
        def _():
            poff = cbase + (q + 1) * NBUF
            pltpu.make_async_copy(src_hbm.at[pl.ds(poff, NBUF)], sb.at[pn],
                                  ps).wait()
            pltpu.make_async_copy(dst_hbm.at[pl.ds(poff, NBUF)], db.at[pn],
                                  pd).wait()
            pltpu.make_async_copy(w_hbm.at[pl.ds(poff, NBUF)], wb.at[pn],
                                  pw).wait()

            @pl.when(q < nq - 2)
            def _():
                poff2 = cbase + (q + 2) * NBUF
                pltpu.async_copy(src_hbm.at[pl.ds(poff2, NBUF)], sb.at[p],
                                 ps)
                pltpu.async_copy(dst_hbm.at[pl.ds(poff2, NBUF)], db.at[p],
                                 pd)
                pltpu.async_copy(w_hbm.at[pl.ds(poff2, NBUF)], wb.at[p],
                                 pw)

            for j in range(NBUF):
                pltpu.async_copy(y_hbm.at[sb.at[pn, j]], rows[j], gsem[j])
        return 0
    lax.fori_loop(0, nq, round_, 0)

    plsc.subcore_barrier()
    pltpu.sync_copy(acc.at[pl.ds(rbase, RPT)],
                    out_hbm.at[cid, pl.ds(rbase, RPT)])


def _make_agg(d, n0):
    return pl.kernel(
        functools.partial(_agg_body, d=d, n0=n0),
        out_type=jax.ShapeDtypeStruct((NC, NPAD, d), jnp.float32),
        mesh=_mesh,
        scratch_types=[
            pltpu.VMEM_SHARED((NPAD, d), jnp.float32),
            pltpu.VMEM((2, NBUF, CH), jnp.int32),
            pltpu.VMEM((2, NBUF, CH), jnp.int32),
            pltpu.VMEM((2, NBUF, CH), jnp.float32),
            *[pltpu.VMEM((CH, d), jnp.float32) for _ in range(NBUF)],
            *[pltpu.SemaphoreType.DMA for _ in range(2 * NBUF + 3)],
        ],
        compiler_params=_sc_params,
    )


_agg_h = _make_agg(D_H, 188)
_agg_o = _make_agg(D_OUT, 160)


BM = 1024
GRID = NPAD // BM


def _tc1_body(x_ref, w_ref, d0_ref, d1_ref, y_ref, dinv_ref):
    deg = d0_ref[...] + d1_ref[...] + 1.0
    dinv = lax.rsqrt(deg)
    xw = jnp.dot(x_ref[...], w_ref[...], preferred_element_type=jnp.float32)
    y_ref[...] = xw * dinv
    dinv_ref[...] = dinv


def _tc1(x, W1, d0, d1):
    return pl.pallas_call(
        _tc1_body,
        grid=(GRID,),
        in_specs=[
            pl.BlockSpec((BM, D_IN), lambda i: (i, 0)),
            pl.BlockSpec((D_IN, D_H), lambda i: (0, 0)),
            pl.BlockSpec((BM, 1), lambda i: (i, 0)),
            pl.BlockSpec((BM, 1), lambda i: (i, 0)),
        ],
        out_specs=[
            pl.BlockSpec((BM, D_H), lambda i: (i, 0)),
            pl.BlockSpec((BM, 1), lambda i: (i, 0)),
        ],
        out_shape=[
            jax.ShapeDtypeStruct((N, D_H), jnp.float32),
            jax.ShapeDtypeStruct((NPAD, 1), jnp.float32),
        ],
    )(x, W1, d0, d1)


def _tc2_body(p_ref, y1_ref, dinv_ref, b1_ref, w2_ref, y2_ref):
    agg = p_ref[0] + p_ref[1] + y1_ref[...]
    h = jnp.maximum(dinv_ref[...] * agg + b1_ref[...], 0.0)
    y2_ref[...] = jnp.dot(h, w2_ref[...],
                          preferred_element_type=jnp.float32) * dinv_ref[...]


def _tc2(p1, y1, dinv, b1, W2):
    return pl.pallas_call(
        _tc2_body,
        grid=(GRID,),
        in_specs=[
            pl.BlockSpec((NC, BM, D_H), lambda i: (0, i, 0)),
            pl.BlockSpec((BM, D_H), lambda i: (i, 0)),
            pl.BlockSpec((BM, 1), lambda i: (i, 0)),
            pl.BlockSpec((1, D_H), lambda i: (0, 0)),
            pl.BlockSpec((D_H, D_OUT), lambda i: (0, 0)),
        ],
        out_specs=pl.BlockSpec((BM, D_OUT), lambda i: (i, 0)),
        out_shape=jax.ShapeDtypeStruct((N, D_OUT), jnp.float32),
    )(p1, y1, dinv, b1, W2)


def _tc3_body(p_ref, y2_ref, dinv_ref, b2_ref, o_ref):
    agg = p_ref[0] + p_ref[1] + y2_ref[...]
    o = dinv_ref[...] * agg + b2_ref[...]
    m = jnp.max(o, axis=-1, keepdims=True)
    e = jnp.exp(o - m)
    lse = jnp.log(jnp.sum(e, axis=-1, keepdims=True))
    o_ref[...] = (o - m) - lse


def _tc3(p2, y2, dinv, b2):
    return pl.pallas_call(
        _tc3_body,
        grid=(GRID,),
        in_specs=[
            pl.BlockSpec((NC, BM, D_OUT), lambda i: (0, i, 0)),
            pl.BlockSpec((BM, D_OUT), lambda i: (i, 0)),
            pl.BlockSpec((BM, 1), lambda i: (i, 0)),
            pl.BlockSpec((1, D_OUT), lambda i: (0, 0)),
        ],
        out_specs=pl.BlockSpec((BM, D_OUT), lambda i: (i, 0)),
        out_shape=jax.ShapeDtypeStruct((N, D_OUT), jnp.float32),
    )(p2, y2, dinv, b2)


def kernel(x, edge_index, edge_weight, W1, b1, W2, b2):
    src = edge_index[0].astype(jnp.int32)
    dst = edge_index[1].astype(jnp.int32)
    ew = edge_weight.astype(jnp.float32)
    npad_e = EPAD - E
    src = jnp.concatenate([src, jnp.zeros((npad_e,), jnp.int32)])
    dst = jnp.concatenate([dst, jnp.zeros((npad_e,), jnp.int32)])
    ew = jnp.concatenate([ew, jnp.zeros((npad_e,), jnp.float32)])

    deg0, deg1, _ = _deg_call(dst, ew)
    d0 = deg0.reshape(NPAD, 1)
    d1 = deg1.reshape(NPAD, 1)

    y1, dinv = _tc1(x, W1, d0, d1)

    src3 = src.reshape(NCH_TOT, CH)
    dst3 = dst.reshape(NCH_TOT, CH)
    ew3 = ew.reshape(NCH_TOT, CH)

    p1 = _agg_h(y1, src3, dst3, ew3)

    y2 = _tc2(p1, y1, dinv, b1.reshape(1, D_H), W2)

    p2 = _agg_o(y2, src3, dst3, ew3)

    return _tc3(p2, y2, dinv, b2.reshape(1, D_OUT))


# speedup vs baseline: 2.3879x; 1.8620x over previous
import functools

import jax
import jax.numpy as jnp
from jax import lax
from jax.experimental import pallas as pl
from jax.experimental.pallas import tpu as pltpu
from jax.experimental.pallas import tpu_sc as plsc

N = 10000
NPAD = 10240
E = 320000
D_IN, D_H, D_OUT = 128, 128, 64

NC, NS = 2, 16
NW = NC * NS
EPAD = 327680
EPT = EPAD // NW
CH = 80
NCHUNK = EPT // CH
RPT = NPAD // NS

_mesh = plsc.VectorSubcoreMesh(core_axis_name="c", subcore_axis_name="s")
_sc_params = pltpu.CompilerParams(needs_layout_passes=False,
                                  use_tc_tiling_on_sc=False)


def _deg_body(dst_hbm, w_hbm, deg0_hbm, deg1_hbm, part_hbm,
              degv, idxb, wb, accb, tmp):
    cid = lax.axis_index("c")
    sid = lax.axis_index("s")
    wid = cid * NS + sid

    def zero(i, _):
        degv[pl.ds(i * 16, 16)] = jnp.zeros((16,), jnp.float32)
        return 0
    lax.fori_loop(0, NPAD // 16, zero, 0)

    ebase = wid * EPT
    pltpu.sync_copy(dst_hbm.at[pl.ds(ebase, EPT)], idxb)
    pltpu.sync_copy(w_hbm.at[pl.ds(ebase, EPT)], wb)

    def inner(i, _):
        sl = pl.ds(i * 16, 16)
        plsc.addupdate_scatter(degv, [idxb[sl]], wb[sl])
        return 0
    lax.fori_loop(0, EPT // 16, inner, 0)

    pltpu.sync_copy(degv, part_hbm.at[cid, sid])
    plsc.subcore_barrier()
    rbase = sid * RPT
    pltpu.sync_copy(part_hbm.at[cid, 0, pl.ds(rbase, RPT)], accb)

    def red(t, _):
        pltpu.sync_copy(part_hbm.at[cid, t, pl.ds(rbase, RPT)], tmp)

        def addv(i, _):
            sl = pl.ds(i * 16, 16)
            accb[sl] = accb[sl] + tmp[sl]
            return 0
        lax.fori_loop(0, RPT // 16, addv, 0)
        return 0
    lax.fori_loop(1, NS, red, 0)

    @pl.when(cid == 0)
    def _():
        pltpu.sync_copy(accb, deg0_hbm.at[pl.ds(rbase, RPT)])

    @pl.when(cid == 1)
    def _():
        pltpu.sync_copy(accb, deg1_hbm.at[pl.ds(rbase, RPT)])


_deg_call = functools.partial(
    pl.kernel,
    out_type=(jax.ShapeDtypeStruct((NPAD,), jnp.float32),
              jax.ShapeDtypeStruct((NPAD,), jnp.float32),
              jax.ShapeDtypeStruct((NC, NS, NPAD), jnp.float32)),
    mesh=_mesh,
    scratch_types=[
        pltpu.VMEM((NPAD,), jnp.float32),
        pltpu.VMEM((EPT,), jnp.int32),
        pltpu.VMEM((EPT,), jnp.float32),
        pltpu.VMEM((RPT,), jnp.float32),
        pltpu.VMEM((RPT,), jnp.float32),
    ],
    compiler_params=_sc_params,
)(_deg_body)


NBUF = 4
NQ = NCHUNK // NBUF


NCH_TOT = EPAD // CH
NTAB = N // NS


def _agg_body(y_hbm, src_hbm, dst_hbm, w_hbm, out_hbm,
              acc, ytab, sb, db, wb, *bufs_and_sems, d, col_split):
    rows = bufs_and_sems[:NBUF]
    gsem = bufs_and_sems[NBUF:2 * NBUF]
    ssem = bufs_and_sems[2 * NBUF:3 * NBUF]
    ps, pd, pw = bufs_and_sems[3 * NBUF:3 * NBUF + 3]
    cid = lax.axis_index("c")
    sid = lax.axis_index("s")
    wid = cid * NS + sid
    rbase = sid * RPT
    npt = NCH_TOT // NS if col_split else NCH_TOT // NW
    nq = npt // NBUF
    cbase = (sid if col_split else wid) * npt

    pltpu.sync_copy(src_hbm.at[pl.ds(cbase, NBUF)], sb.at[0])
    pltpu.sync_copy(dst_hbm.at[pl.ds(cbase, NBUF)], db.at[0])
    pltpu.sync_copy(w_hbm.at[pl.ds(cbase, NBUF)], wb.at[0])
    pltpu.async_copy(src_hbm.at[pl.ds(cbase + NBUF, NBUF)], sb.at[1], ps)
    pltpu.async_copy(dst_hbm.at[pl.ds(cbase + NBUF, NBUF)], db.at[1], pd)
    pltpu.async_copy(w_hbm.at[pl.ds(cbase + NBUF, NBUF)], wb.at[1], pw)

    tslice = pl.ds(sid * NTAB, NTAB)
    if col_split:
        pltpu.sync_copy(y_hbm.at[cid, tslice], ytab.at[tslice])
    else:
        pltpu.sync_copy(y_hbm.at[tslice], ytab.at[tslice])

    def zrow(r, _):
        for jv in range(d // 16):
            rows[0][r, pl.ds(jv * 16, 16)] = jnp.zeros((16,), jnp.float32)
        return 0
    lax.fori_loop(0, CH, zrow, 0)
    for t in range(RPT // CH):
        pltpu.sync_copy(rows[0], acc.at[pl.ds(rbase + t * CH, CH)])

    plsc.subcore_barrier()

    for j in range(NBUF):
        pltpu.async_copy(ytab.at[sb.at[0, j]], rows[j], gsem[j])

    def round_(q, _):
        p = lax.rem(q, 2)
        pn = 1 - p
        for j in range(NBUF):
            with jax.named_scope("gwait"):
                pltpu.make_async_copy(ytab.at[sb.at[p, j]], rows[j],
                                      gsem[j]).wait()

            with jax.named_scope("scale"):
                @plsc.parallel_loop(0, CH // 16, unroll=2)
                def scale(g):
                    w16 = wb[p, j, pl.ds(g * 16, 16)]
                    for e in range(16):
                        we = w16[e]
                        r = g * 16 + e
                        for jv in range(d // 16):
                            sl = pl.ds(jv * 16, 16)
                            rows[j][r, sl] = rows[j][r, sl] * we

            pltpu.async_copy(rows[j], acc.at[db.at[p, j]], ssem[j], add=True)
        with jax.named_scope("swait"):
            for j in range(NBUF):
                pltpu.make_async_copy(rows[j], acc.at[db.at[p, j]],
                                      ssem[j]).wait()

        @pl.when(q < nq - 1)
        def _():
            poff = cbase + (q + 1) * NBUF
            pltpu.make_async_copy(src_hbm.at[pl.ds(poff, NBUF)], sb.at[pn],
                                  ps).wait()
            pltpu.make_async_copy(dst_hbm.at[pl.ds(poff, NBUF)], db.at[pn],
                                  pd).wait()
            pltpu.make_async_copy(w_hbm.at[pl.ds(poff, NBUF)], wb.at[pn],
                                  pw).wait()

            @pl.when(q < nq - 2)
            def _():
                poff2 = cbase + (q + 2) * NBUF
                pltpu.async_copy(src_hbm.at[pl.ds(poff2, NBUF)], sb.at[p],
                                 ps)
                pltpu.async_copy(dst_hbm.at[pl.ds(poff2, NBUF)], db.at[p],
                                 pd)
                pltpu.async_copy(w_hbm.at[pl.ds(poff2, NBUF)], wb.at[p],
                                 pw)

            for j in range(NBUF):
                pltpu.async_copy(ytab.at[sb.at[pn, j]], rows[j], gsem[j])
        return 0
    lax.fori_loop(0, nq, round_, 0)

    plsc.subcore_barrier()
    pltpu.sync_copy(acc.at[pl.ds(rbase, RPT)],
                    out_hbm.at[cid, pl.ds(rbase, RPT)])


def _make_agg(d, col_split):
    return pl.kernel(
        functools.partial(_agg_body, d=d, col_split=col_split),
        out_type=jax.ShapeDtypeStruct((NC, NPAD, d), jnp.float32),
        mesh=_mesh,
        scratch_types=[
            pltpu.VMEM_SHARED((NPAD, d), jnp.float32),
            pltpu.VMEM_SHARED((N, d), jnp.float32),
            pltpu.VMEM((2, NBUF, CH), jnp.int32),
            pltpu.VMEM((2, NBUF, CH), jnp.int32),
            pltpu.VMEM((2, NBUF, CH), jnp.float32),
            *[pltpu.VMEM((CH, d), jnp.float32) for _ in range(NBUF)],
            *[pltpu.SemaphoreType.DMA for _ in range(2 * NBUF + 3)],
        ],
        compiler_params=_sc_params,
    )


_agg_h = _make_agg(D_H // 2, True)
_agg_o = _make_agg(D_OUT, False)


BM = 1024
GRID = NPAD // BM


def _tc1_body(x_ref, w_ref, d0_ref, d1_ref, y_ref, dinv_ref):
    deg = d0_ref[...] + d1_ref[...] + 1.0
    dinv = lax.rsqrt(deg)
    xw = jnp.dot(x_ref[...], w_ref[...], preferred_element_type=jnp.float32)
    y = xw * dinv
    y_ref[0] = y[:, :D_H // 2]
    y_ref[1] = y[:, D_H // 2:]
    dinv_ref[...] = dinv


def _tc1(x, W1, d0, d1):
    return pl.pallas_call(
        _tc1_body,
        grid=(GRID,),
        in_specs=[
            pl.BlockSpec((BM, D_IN), lambda i: (i, 0)),
            pl.BlockSpec((D_IN, D_H), lambda i: (0, 0)),
            pl.BlockSpec((BM, 1), lambda i: (i, 0)),
            pl.BlockSpec((BM, 1), lambda i: (i, 0)),
        ],
        out_specs=[
            pl.BlockSpec((NC, BM, D_H // 2), lambda i: (0, i, 0)),
            pl.BlockSpec((BM, 1), lambda i: (i, 0)),
        ],
        out_shape=[
            jax.ShapeDtypeStruct((NC, N, D_H // 2), jnp.float32),
            jax.ShapeDtypeStruct((NPAD, 1), jnp.float32),
        ],
    )(x, W1, d0, d1)


def _tc2_body(p_ref, y1_ref, dinv_ref, b1_ref, w2_ref, y2_ref):
    agg = jnp.concatenate(
        [p_ref[0] + y1_ref[0], p_ref[1] + y1_ref[1]], axis=-1)
    h = jnp.maximum(dinv_ref[...] * agg + b1_ref[...], 0.0)
    y2_ref[...] = jnp.dot(h, w2_ref[...],
                          preferred_element_type=jnp.float32) * dinv_ref[...]


def _tc2(p1, y1, dinv, b1, W2):
    return pl.pallas_call(
        _tc2_body,
        grid=(GRID,),
        in_specs=[
            pl.BlockSpec((NC, BM, D_H // 2), lambda i: (0, i, 0)),
            pl.BlockSpec((NC, BM, D_H // 2), lambda i: (0, i, 0)),
            pl.BlockSpec((BM, 1), lambda i: (i, 0)),
            pl.BlockSpec((1, D_H), lambda i: (0, 0)),
            pl.BlockSpec((D_H, D_OUT), lambda i: (0, 0)),
        ],
        out_specs=pl.BlockSpec((BM, D_OUT), lambda i: (i, 0)),
        out_shape=jax.ShapeDtypeStruct((N, D_OUT), jnp.float32),
    )(p1, y1, dinv, b1, W2)


def _tc3_body(p_ref, y2_ref, dinv_ref, b2_ref, o_ref):
    agg = p_ref[0] + p_ref[1] + y2_ref[...]
    o = dinv_ref[...] * agg + b2_ref[...]
    m = jnp.max(o, axis=-1, keepdims=True)
    e = jnp.exp(o - m)
    lse = jnp.log(jnp.sum(e, axis=-1, keepdims=True))
    o_ref[...] = (o - m) - lse


def _tc3(p2, y2, dinv, b2):
    return pl.pallas_call(
        _tc3_body,
        grid=(GRID,),
        in_specs=[
            pl.BlockSpec((NC, BM, D_OUT), lambda i: (0, i, 0)),
            pl.BlockSpec((BM, D_OUT), lambda i: (i, 0)),
            pl.BlockSpec((BM, 1), lambda i: (i, 0)),
            pl.BlockSpec((1, D_OUT), lambda i: (0, 0)),
        ],
        out_specs=pl.BlockSpec((BM, D_OUT), lambda i: (i, 0)),
        out_shape=jax.ShapeDtypeStruct((N, D_OUT), jnp.float32),
    )(p2, y2, dinv, b2)


def kernel(x, edge_index, edge_weight, W1, b1, W2, b2):
    src = edge_index[0].astype(jnp.int32)
    dst = edge_index[1].astype(jnp.int32)
    ew = edge_weight.astype(jnp.float32)
    npad_e = EPAD - E
    src = jnp.concatenate([src, jnp.zeros((npad_e,), jnp.int32)])
    dst = jnp.concatenate([dst, jnp.zeros((npad_e,), jnp.int32)])
    ew = jnp.concatenate([ew, jnp.zeros((npad_e,), jnp.float32)])

    deg0, deg1, _ = _deg_call(dst, ew)
    d0 = deg0.reshape(NPAD, 1)
    d1 = deg1.reshape(NPAD, 1)

    y1, dinv = _tc1(x, W1, d0, d1)

    src3 = src.reshape(NCH_TOT, CH)
    dst3 = dst.reshape(NCH_TOT, CH)
    ew3 = ew.reshape(NCH_TOT, CH)

    p1 = _agg_h(y1, src3, dst3, ew3)

    y2 = _tc2(p1, y1, dinv, b1.reshape(1, D_H), W2)

    p2 = _agg_o(y2, src3, dst3, ew3)

    return _tc3(p2, y2, dinv, b2.reshape(1, D_OUT))


# speedup vs baseline: 2.4495x; 1.0258x over previous
import functools

import jax
import jax.numpy as jnp
from jax import lax
from jax.experimental import pallas as pl
from jax.experimental.pallas import tpu as pltpu
from jax.experimental.pallas import tpu_sc as plsc

N = 10000
NPAD = 10240
E = 320000
D_IN, D_H, D_OUT = 128, 128, 64

NC, NS = 2, 16
NW = NC * NS
EPAD = 327680
EPT = EPAD // NW
CH = 80
NCHUNK = EPT // CH
RPT = NPAD // NS

_mesh = plsc.VectorSubcoreMesh(core_axis_name="c", subcore_axis_name="s")
_sc_params = pltpu.CompilerParams(needs_layout_passes=False,
                                  use_tc_tiling_on_sc=False)


def _deg_body(dst_hbm, w_hbm, deg0_hbm, deg1_hbm, part_hbm,
              degv, idxb, wb, accb, tmp):
    cid = lax.axis_index("c")
    sid = lax.axis_index("s")
    wid = cid * NS + sid

    def zero(i, _):
        degv[pl.ds(i * 16, 16)] = jnp.zeros((16,), jnp.float32)
        return 0
    lax.fori_loop(0, NPAD // 16, zero, 0)

    ebase = wid * EPT
    pltpu.sync_copy(dst_hbm.at[pl.ds(ebase, EPT)], idxb)
    pltpu.sync_copy(w_hbm.at[pl.ds(ebase, EPT)], wb)

    def inner(i, _):
        sl = pl.ds(i * 16, 16)
        plsc.addupdate_scatter(degv, [idxb[sl]], wb[sl])
        return 0
    lax.fori_loop(0, EPT // 16, inner, 0)

    pltpu.sync_copy(degv, part_hbm.at[cid, sid])
    plsc.subcore_barrier()
    rbase = sid * RPT
    pltpu.sync_copy(part_hbm.at[cid, 0, pl.ds(rbase, RPT)], accb)

    def red(t, _):
        pltpu.sync_copy(part_hbm.at[cid, t, pl.ds(rbase, RPT)], tmp)

        def addv(i, _):
            sl = pl.ds(i * 16, 16)
            accb[sl] = accb[sl] + tmp[sl]
            return 0
        lax.fori_loop(0, RPT // 16, addv, 0)
        return 0
    lax.fori_loop(1, NS, red, 0)

    @pl.when(cid == 0)
    def _():
        pltpu.sync_copy(accb, deg0_hbm.at[pl.ds(rbase, RPT)])

    @pl.when(cid == 1)
    def _():
        pltpu.sync_copy(accb, deg1_hbm.at[pl.ds(rbase, RPT)])


_deg_call = functools.partial(
    pl.kernel,
    out_type=(jax.ShapeDtypeStruct((NPAD,), jnp.float32),
              jax.ShapeDtypeStruct((NPAD,), jnp.float32),
              jax.ShapeDtypeStruct((NC, NS, NPAD), jnp.float32)),
    mesh=_mesh,
    scratch_types=[
        pltpu.VMEM((NPAD,), jnp.float32),
        pltpu.VMEM((EPT,), jnp.int32),
        pltpu.VMEM((EPT,), jnp.float32),
        pltpu.VMEM((RPT,), jnp.float32),
        pltpu.VMEM((RPT,), jnp.float32),
    ],
    compiler_params=_sc_params,
)(_deg_body)


NBUF = 4
NQ = NCHUNK // NBUF


NCH_TOT = EPAD // CH
NTAB = N // NS


def _agg_body(y_hbm, src_hbm, dst_hbm, w_hbm, out_hbm,
              acc, ytab, sb, db, wb, *bufs_and_sems, d, col_split):
    rows = bufs_and_sems[:NBUF]
    gsem = bufs_and_sems[NBUF:2 * NBUF]
    ssem = bufs_and_sems[2 * NBUF:3 * NBUF]
    ps, pd, pw = bufs_and_sems[3 * NBUF:3 * NBUF + 3]
    cid = lax.axis_index("c")
    sid = lax.axis_index("s")
    wid = cid * NS + sid
    rbase = sid * RPT
    if col_split:
        npt = NCH_TOT // NS
        nq = npt // NBUF
        cbase = sid * npt
    else:
        n0 = 148
        n1 = NCH_TOT // NS - n0
        nq = jnp.where(cid == 0, n0 // NBUF, n1 // NBUF)
        cbase = jnp.where(cid == 0, sid * n0, NS * n0 + sid * n1)

    pltpu.sync_copy(src_hbm.at[pl.ds(cbase, NBUF)], sb.at[0])
    pltpu.sync_copy(dst_hbm.at[pl.ds(cbase, NBUF)], db.at[0])
    pltpu.sync_copy(w_hbm.at[pl.ds(cbase, NBUF)], wb.at[0])
    pltpu.async_copy(src_hbm.at[pl.ds(cbase + NBUF, NBUF)], sb.at[1], ps)
    pltpu.async_copy(dst_hbm.at[pl.ds(cbase + NBUF, NBUF)], db.at[1], pd)
    pltpu.async_copy(w_hbm.at[pl.ds(cbase + NBUF, NBUF)], wb.at[1], pw)

    tslice = pl.ds(sid * NTAB, NTAB)
    if col_split:
        pltpu.sync_copy(y_hbm.at[cid, tslice], ytab.at[tslice])
    else:
        pltpu.sync_copy(y_hbm.at[tslice], ytab.at[tslice])

    def zrow(r, _):
        for jv in range(d // 16):
            rows[0][r, pl.ds(jv * 16, 16)] = jnp.zeros((16,), jnp.float32)
        return 0
    lax.fori_loop(0, CH, zrow, 0)
    for t in range(RPT // CH):
        pltpu.sync_copy(rows[0], acc.at[pl.ds(rbase + t * CH, CH)])

    plsc.subcore_barrier()

    for j in range(NBUF):
        pltpu.async_copy(ytab.at[sb.at[0, j]], rows[j], gsem[j])

    def round_(q, _):
        p = lax.rem(q, 2)
        pn = 1 - p
        for j in range(NBUF):
            with jax.named_scope("gwait"):
                pltpu.make_async_copy(ytab.at[sb.at[p, j]], rows[j],
                                      gsem[j]).wait()

            with jax.named_scope("scale"):
                @plsc.parallel_loop(0, CH // 16, unroll=2)
                def scale(g):
                    w16 = wb[p, j, pl.ds(g * 16, 16)]
                    for e in range(16):
                        we = w16[e]
                        r = g * 16 + e
                        for jv in range(d // 16):
                            sl = pl.ds(jv * 16, 16)
                            rows[j][r, sl] = rows[j][r, sl] * we

            pltpu.async_copy(rows[j], acc.at[db.at[p, j]], ssem[j], add=True)
        with jax.named_scope("swait"):
            for j in range(NBUF):
                pltpu.make_async_copy(rows[j], acc.at[db.at[p, j]],
                                      ssem[j]).wait()

        @pl.when(q < nq - 1)
        def _():
            poff = cbase + (q + 1) * NBUF
            pltpu.make_async_copy(src_hbm.at[pl.ds(poff, NBUF)], sb.at[pn],
                                  ps).wait()
            pltpu.make_async_copy(dst_hbm.at[pl.ds(poff, NBUF)], db.at[pn],
                                  pd).wait()
            pltpu.make_async_copy(w_hbm.at[pl.ds(poff, NBUF)], wb.at[pn],
                                  pw).wait()

            @pl.when(q < nq - 2)
            def _():
                poff2 = cbase + (q + 2) * NBUF
                pltpu.async_copy(src_hbm.at[pl.ds(poff2, NBUF)], sb.at[p],
                                 ps)
                pltpu.async_copy(dst_hbm.at[pl.ds(poff2, NBUF)], db.at[p],
                                 pd)
                pltpu.async_copy(w_hbm.at[pl.ds(poff2, NBUF)], wb.at[p],
                                 pw)

            for j in range(NBUF):
                pltpu.async_copy(ytab.at[sb.at[pn, j]], rows[j], gsem[j])
        return 0
    lax.fori_loop(0, nq, round_, 0)

    plsc.subcore_barrier()
    pltpu.sync_copy(acc.at[pl.ds(rbase, RPT)],
                    out_hbm.at[cid, pl.ds(rbase, RPT)])


def _make_agg(d, col_split):
    return pl.kernel(
        functools.partial(_agg_body, d=d, col_split=col_split),
        out_type=jax.ShapeDtypeStruct((NC, NPAD, d), jnp.float32),
        mesh=_mesh,
        scratch_types=[
            pltpu.VMEM_SHARED((NPAD, d), jnp.float32),
            pltpu.VMEM_SHARED((N, d), jnp.float32),
            pltpu.VMEM((2, NBUF, CH), jnp.int32),
            pltpu.VMEM((2, NBUF, CH), jnp.int32),
            pltpu.VMEM((2, NBUF, CH), jnp.float32),
            *[pltpu.VMEM((CH, d), jnp.float32) for _ in range(NBUF)],
            *[pltpu.SemaphoreType.DMA for _ in range(2 * NBUF + 3)],
        ],
        compiler_params=_sc_params,
    )


_agg_h = _make_agg(D_H // 2, True)
_agg_o = _make_agg(D_OUT, False)


BM = 1024
GRID = NPAD // BM


def _tcmm_body(x_ref, w_ref, xw_ref):
    xw_ref[...] = jnp.dot(x_ref[...], w_ref[...],
                          preferred_element_type=jnp.float32)


def _tcmm(x, W1):
    return pl.pallas_call(
        _tcmm_body,
        grid=(GRID,),
        in_specs=[
            pl.BlockSpec((BM, D_IN), lambda i: (i, 0)),
            pl.BlockSpec((D_IN, D_H), lambda i: (0, 0)),
        ],
        out_specs=pl.BlockSpec((BM, D_H), lambda i: (i, 0)),
        out_shape=jax.ShapeDtypeStruct((N, D_H), jnp.float32),
    )(x, W1)


def _tc1_body(xw_ref, d0_ref, d1_ref, y_ref, dinv_ref):
    deg = d0_ref[...] + d1_ref[...] + 1.0
    dinv = lax.rsqrt(deg)
    y = xw_ref[...] * dinv
    y_ref[0] = y[:, :D_H // 2]
    y_ref[1] = y[:, D_H // 2:]
    dinv_ref[...] = dinv


def _tc1(xw, d0, d1):
    return pl.pallas_call(
        _tc1_body,
        grid=(GRID,),
        in_specs=[
            pl.BlockSpec((BM, D_H), lambda i: (i, 0)),
            pl.BlockSpec((BM, 1), lambda i: (i, 0)),
            pl.BlockSpec((BM, 1), lambda i: (i, 0)),
        ],
        out_specs=[
            pl.BlockSpec((NC, BM, D_H // 2), lambda i: (0, i, 0)),
            pl.BlockSpec((BM, 1), lambda i: (i, 0)),
        ],
        out_shape=[
            jax.ShapeDtypeStruct((NC, N, D_H // 2), jnp.float32),
            jax.ShapeDtypeStruct((NPAD, 1), jnp.float32),
        ],
    )(xw, d0, d1)


def _tc2_body(p_ref, y1_ref, dinv_ref, b1_ref, w2_ref, y2_ref):
    agg = jnp.concatenate(
        [p_ref[0] + y1_ref[0], p_ref[1] + y1_ref[1]], axis=-1)
    h = jnp.maximum(dinv_ref[...] * agg + b1_ref[...], 0.0)
    y2_ref[...] = jnp.dot(h, w2_ref[...],
                          preferred_element_type=jnp.float32) * dinv_ref[...]


def _tc2(p1, y1, dinv, b1, W2):
    return pl.pallas_call(
        _tc2_body,
        grid=(GRID,),
        in_specs=[
            pl.BlockSpec((NC, BM, D_H // 2), lambda i: (0, i, 0)),
            pl.BlockSpec((NC, BM, D_H // 2), lambda i: (0, i, 0)),
            pl.BlockSpec((BM, 1), lambda i: (i, 0)),
            pl.BlockSpec((1, D_H), lambda i: (0, 0)),
            pl.BlockSpec((D_H, D_OUT), lambda i: (0, 0)),
        ],
        out_specs=pl.BlockSpec((BM, D_OUT), lambda i: (i, 0)),
        out_shape=jax.ShapeDtypeStruct((N, D_OUT), jnp.float32),
    )(p1, y1, dinv, b1, W2)


def _tc3_body(p_ref, y2_ref, dinv_ref, b2_ref, o_ref):
    agg = p_ref[0] + p_ref[1] + y2_ref[...]
    o = dinv_ref[...] * agg + b2_ref[...]
    m = jnp.max(o, axis=-1, keepdims=True)
    e = jnp.exp(o - m)
    lse = jnp.log(jnp.sum(e, axis=-1, keepdims=True))
    o_ref[...] = (o - m) - lse


def _tc3(p2, y2, dinv, b2):
    return pl.pallas_call(
        _tc3_body,
        grid=(GRID,),
        in_specs=[
            pl.BlockSpec((NC, BM, D_OUT), lambda i: (0, i, 0)),
            pl.BlockSpec((BM, D_OUT), lambda i: (i, 0)),
            pl.BlockSpec((BM, 1), lambda i: (i, 0)),
            pl.BlockSpec((1, D_OUT), lambda i: (0, 0)),
        ],
        out_specs=pl.BlockSpec((BM, D_OUT), lambda i: (i, 0)),
        out_shape=jax.ShapeDtypeStruct((N, D_OUT), jnp.float32),
    )(p2, y2, dinv, b2)


def kernel(x, edge_index, edge_weight, W1, b1, W2, b2):
    src = edge_index[0].astype(jnp.int32)
    dst = edge_index[1].astype(jnp.int32)
    ew = edge_weight.astype(jnp.float32)
    npad_e = EPAD - E
    src = jnp.concatenate([src, jnp.zeros((npad_e,), jnp.int32)])
    dst = jnp.concatenate([dst, jnp.zeros((npad_e,), jnp.int32)])
    ew = jnp.concatenate([ew, jnp.zeros((npad_e,), jnp.float32)])

    deg0, deg1, _ = _deg_call(dst, ew)
    xw = _tcmm(x, W1)
    d0 = deg0.reshape(NPAD, 1)
    d1 = deg1.reshape(NPAD, 1)

    y1, dinv = _tc1(xw, d0, d1)

    src3 = src.reshape(NCH_TOT, CH)
    dst3 = dst.reshape(NCH_TOT, CH)
    ew3 = ew.reshape(NCH_TOT, CH)

    p1 = _agg_h(y1, src3, dst3, ew3)

    y2 = _tc2(p1, y1, dinv, b1.reshape(1, D_H), W2)

    p2 = _agg_o(y2, src3, dst3, ew3)

    return _tc3(p2, y2, dinv, b2.reshape(1, D_OUT))


# speedup vs baseline: 2.7322x; 1.1154x over previous
import functools

import jax
import jax.numpy as jnp
from jax import lax
from jax.experimental import pallas as pl
from jax.experimental.pallas import tpu as pltpu
from jax.experimental.pallas import tpu_sc as plsc

N = 10000
NPAD = 10240
E = 320000
D_IN, D_H, D_OUT = 128, 128, 64

NC, NS = 2, 16
NW = NC * NS
EPAD = 327680
EPT = EPAD // NW
CH = 80
NCHUNK = EPT // CH
RPT = NPAD // NS

_mesh = plsc.VectorSubcoreMesh(core_axis_name="c", subcore_axis_name="s")
_sc_params = pltpu.CompilerParams(needs_layout_passes=False,
                                  use_tc_tiling_on_sc=False)


def _deg_body(dst_hbm, w_hbm, deg0_hbm, deg1_hbm,
              degv, idxb, wb, accb, tmp, sdeg):
    cid = lax.axis_index("c")
    sid = lax.axis_index("s")
    wid = cid * NS + sid

    def zero(i, _):
        degv[pl.ds(i * 16, 16)] = jnp.zeros((16,), jnp.float32)
        return 0
    lax.fori_loop(0, NPAD // 16, zero, 0)

    ebase = wid * EPT
    pltpu.sync_copy(dst_hbm.at[pl.ds(ebase, EPT)], idxb)
    pltpu.sync_copy(w_hbm.at[pl.ds(ebase, EPT)], wb)

    def inner(i, _):
        sl = pl.ds(i * 16, 16)
        plsc.addupdate_scatter(degv, [idxb[sl]], wb[sl])
        return 0
    lax.fori_loop(0, EPT // 16, inner, 0)

    pltpu.sync_copy(degv, sdeg.at[sid])
    plsc.subcore_barrier()
    rbase = sid * RPT
    pltpu.sync_copy(sdeg.at[0, pl.ds(rbase, RPT)], accb)

    def red(t, _):
        pltpu.sync_copy(sdeg.at[t, pl.ds(rbase, RPT)], tmp)

        def addv(i, _):
            sl = pl.ds(i * 16, 16)
            accb[sl] = accb[sl] + tmp[sl]
            return 0
        lax.fori_loop(0, RPT // 16, addv, 0)
        return 0
    lax.fori_loop(1, NS, red, 0)

    @pl.when(cid == 0)
    def _():
        pltpu.sync_copy(accb, deg0_hbm.at[pl.ds(rbase, RPT)])

    @pl.when(cid == 1)
    def _():
        pltpu.sync_copy(accb, deg1_hbm.at[pl.ds(rbase, RPT)])


_deg_call = functools.partial(
    pl.kernel,
    out_type=(jax.ShapeDtypeStruct((NPAD,), jnp.float32),
              jax.ShapeDtypeStruct((NPAD,), jnp.float32)),
    mesh=_mesh,
    scratch_types=[
        pltpu.VMEM((NPAD,), jnp.float32),
        pltpu.VMEM((EPT,), jnp.int32),
        pltpu.VMEM((EPT,), jnp.float32),
        pltpu.VMEM((RPT,), jnp.float32),
        pltpu.VMEM((RPT,), jnp.float32),
        pltpu.VMEM_SHARED((NS, NPAD), jnp.float32),
    ],
    compiler_params=_sc_params,
)(_deg_body)


NBUF = 4
NQ = NCHUNK // NBUF


NCH_TOT = EPAD // CH
NTAB = N // NS


def _agg_body(y_hbm, src_hbm, dst_hbm, w_hbm, out_hbm,
              acc, ytab, sb, db, wb, *bufs_and_sems, d, col_split):
    rows = bufs_and_sems[:NBUF]
    gsem = bufs_and_sems[NBUF:2 * NBUF]
    ssem = bufs_and_sems[2 * NBUF:3 * NBUF]
    ps, pd, pw = bufs_and_sems[3 * NBUF:3 * NBUF + 3]
    cid = lax.axis_index("c")
    sid = lax.axis_index("s")
    wid = cid * NS + sid
    rbase = sid * RPT
    if col_split:
        npt = NCH_TOT // NS
        nq = npt // NBUF
        cbase = sid * npt
    else:
        n0 = 148
        n1 = NCH_TOT // NS - n0
        nq = jnp.where(cid == 0, n0 // NBUF, n1 // NBUF)
        cbase = jnp.where(cid == 0, sid * n0, NS * n0 + sid * n1)

    pltpu.sync_copy(src_hbm.at[pl.ds(cbase, NBUF)], sb.at[0])
    pltpu.sync_copy(dst_hbm.at[pl.ds(cbase, NBUF)], db.at[0])
    pltpu.sync_copy(w_hbm.at[pl.ds(cbase, NBUF)], wb.at[0])
    pltpu.async_copy(src_hbm.at[pl.ds(cbase + NBUF, NBUF)], sb.at[1], ps)
    pltpu.async_copy(dst_hbm.at[pl.ds(cbase + NBUF, NBUF)], db.at[1], pd)
    pltpu.async_copy(w_hbm.at[pl.ds(cbase + NBUF, NBUF)], wb.at[1], pw)

    tslice = pl.ds(sid * NTAB, NTAB)
    coff = cid * d if col_split else 0
    pltpu.sync_copy(y_hbm.at[tslice, pl.ds(coff, d)], ytab.at[tslice])

    def zrow(r, _):
        for jv in range(d // 16):
            rows[0][r, pl.ds(jv * 16, 16)] = jnp.zeros((16,), jnp.float32)
        return 0
    lax.fori_loop(0, CH, zrow, 0)
    for t in range(RPT // CH):
        pltpu.sync_copy(rows[0], acc.at[pl.ds(rbase + t * CH, CH)])

    plsc.subcore_barrier()

    for j in range(NBUF):
        pltpu.async_copy(ytab.at[sb.at[0, j]], rows[j], gsem[j])

    def round_(q, _):
        p = lax.rem(q, 2)
        pn = 1 - p
        for j in range(NBUF):
            with jax.named_scope("gwait"):
                pltpu.make_async_copy(ytab.at[sb.at[p, j]], rows[j],
                                      gsem[j]).wait()

            with jax.named_scope("scale"):
                @plsc.parallel_loop(0, CH // 16, unroll=2)
                def scale(g):
                    w16 = wb[p, j, pl.ds(g * 16, 16)]
                    for e in range(16):
                        we = w16[e]
                        r = g * 16 + e
                        for jv in range(d // 16):
                            sl = pl.ds(jv * 16, 16)
                            rows[j][r, sl] = rows[j][r, sl] * we

            pltpu.async_copy(rows[j], acc.at[db.at[p, j]], ssem[j], add=True)
        with jax.named_scope("swait"):
            for j in range(NBUF):
                pltpu.make_async_copy(rows[j], acc.at[db.at[p, j]],
                                      ssem[j]).wait()

        @pl.when(q < nq - 1)
        def _():
            poff = cbase + (q + 1) * NBUF
            pltpu.make_async_copy(src_hbm.at[pl.ds(poff, NBUF)], sb.at[pn],
                                  ps).wait()
            pltpu.make_async_copy(dst_hbm.at[pl.ds(poff, NBUF)], db.at[pn],
                                  pd).wait()
            pltpu.make_async_copy(w_hbm.at[pl.ds(poff, NBUF)], wb.at[pn],
                                  pw).wait()

            @pl.when(q < nq - 2)
            def _():
                poff2 = cbase + (q + 2) * NBUF
                pltpu.async_copy(src_hbm.at[pl.ds(poff2, NBUF)], sb.at[p],
                                 ps)
                pltpu.async_copy(dst_hbm.at[pl.ds(poff2, NBUF)], db.at[p],
                                 pd)
                pltpu.async_copy(w_hbm.at[pl.ds(poff2, NBUF)], wb.at[p],
                                 pw)

            for j in range(NBUF):
                pltpu.async_copy(ytab.at[sb.at[pn, j]], rows[j], gsem[j])
        return 0
    lax.fori_loop(0, nq, round_, 0)

    plsc.subcore_barrier()
    pltpu.sync_copy(acc.at[pl.ds(rbase, RPT)],
                    out_hbm.at[pl.ds(rbase, RPT), pl.ds(cid * d, d)])


def _make_agg(d, col_split):
    return pl.kernel(
        functools.partial(_agg_body, d=d, col_split=col_split),
        out_type=jax.ShapeDtypeStruct((NPAD, 2 * d), jnp.float32),
        mesh=_mesh,
        scratch_types=[
            pltpu.VMEM_SHARED((NPAD, d), jnp.float32),
            pltpu.VMEM_SHARED((N, d), jnp.float32),
            pltpu.VMEM((2, NBUF, CH), jnp.int32),
            pltpu.VMEM((2, NBUF, CH), jnp.int32),
            pltpu.VMEM((2, NBUF, CH), jnp.float32),
            *[pltpu.VMEM((CH, d), jnp.float32) for _ in range(NBUF)],
            *[pltpu.SemaphoreType.DMA for _ in range(2 * NBUF + 3)],
        ],
        compiler_params=_sc_params,
    )


_agg_h = _make_agg(D_H // 2, True)
_agg_o = _make_agg(D_OUT, False)


BM = 1024
GRID = NPAD // BM


def _tcmm_body(x_ref, w_ref, xw_ref):
    xw_ref[...] = jnp.dot(x_ref[...], w_ref[...],
                          preferred_element_type=jnp.float32)


def _tcmm(x, W1):
    return pl.pallas_call(
        _tcmm_body,
        grid=(GRID,),
        in_specs=[
            pl.BlockSpec((BM, D_IN), lambda i: (i, 0)),
            pl.BlockSpec((D_IN, D_H), lambda i: (0, 0)),
        ],
        out_specs=pl.BlockSpec((BM, D_H), lambda i: (i, 0)),
        out_shape=jax.ShapeDtypeStruct((N, D_H), jnp.float32),
    )(x, W1)


def _tc1_body(xw_ref, d0_ref, d1_ref, y_ref, dinv_ref):
    deg = d0_ref[...] + d1_ref[...] + 1.0
    dinv = lax.rsqrt(deg)
    y_ref[...] = xw_ref[...] * dinv
    dinv_ref[...] = dinv


def _tc1(xw, d0, d1):
    return pl.pallas_call(
        _tc1_body,
        grid=(GRID,),
        in_specs=[
            pl.BlockSpec((BM, D_H), lambda i: (i, 0)),
            pl.BlockSpec((BM, 1), lambda i: (i, 0)),
            pl.BlockSpec((BM, 1), lambda i: (i, 0)),
        ],
        out_specs=[
            pl.BlockSpec((BM, D_H), lambda i: (i, 0)),
            pl.BlockSpec((BM, 1), lambda i: (i, 0)),
        ],
        out_shape=[
            jax.ShapeDtypeStruct((N, D_H), jnp.float32),
            jax.ShapeDtypeStruct((NPAD, 1), jnp.float32),
        ],
    )(xw, d0, d1)


def _tc2_body(p_ref, y1_ref, dinv_ref, b1_ref, w2_ref, y2_ref):
    agg = p_ref[...] + y1_ref[...]
    h = jnp.maximum(dinv_ref[...] * agg + b1_ref[...], 0.0)
    y2 = jnp.dot(h, w2_ref[...],
                 preferred_element_type=jnp.float32) * dinv_ref[...]
    y2_ref[...] = jnp.concatenate(
        [y2, jnp.zeros((BM, D_H - D_OUT), jnp.float32)], axis=-1)


def _tc2(p1, y1, dinv, b1, W2):
    return pl.pallas_call(
        _tc2_body,
        grid=(GRID,),
        in_specs=[
            pl.BlockSpec((BM, D_H), lambda i: (i, 0)),
            pl.BlockSpec((BM, D_H), lambda i: (i, 0)),
            pl.BlockSpec((BM, 1), lambda i: (i, 0)),
            pl.BlockSpec((1, D_H), lambda i: (0, 0)),
            pl.BlockSpec((D_H, D_OUT), lambda i: (0, 0)),
        ],
        out_specs=pl.BlockSpec((BM, D_H), lambda i: (i, 0)),
        out_shape=jax.ShapeDtypeStruct((N, D_H), jnp.float32),
    )(p1, y1, dinv, b1, W2)


def _tc3_body(p_ref, y2_ref, dinv_ref, b2_ref, o_ref):
    agg = (p_ref[:, :D_OUT] + p_ref[:, D_OUT:] + y2_ref[:, :D_OUT])
    o = dinv_ref[...] * agg + b2_ref[...]
    m = jnp.max(o, axis=-1, keepdims=True)
    e = jnp.exp(o - m)
    lse = jnp.log(jnp.sum(e, axis=-1, keepdims=True))
    o_ref[...] = (o - m) - lse


def _tc3(p2, y2, dinv, b2):
    return pl.pallas_call(
        _tc3_body,
        grid=(GRID,),
        in_specs=[
            pl.BlockSpec((BM, 2 * D_OUT), lambda i: (i, 0)),
            pl.BlockSpec((BM, D_H), lambda i: (i, 0)),
            pl.BlockSpec((BM, 1), lambda i: (i, 0)),
            pl.BlockSpec((1, D_OUT), lambda i: (0, 0)),
        ],
        out_specs=pl.BlockSpec((BM, D_OUT), lambda i: (i, 0)),
        out_shape=jax.ShapeDtypeStruct((N, D_OUT), jnp.float32),
    )(p2, y2, dinv, b2)


def kernel(x, edge_index, edge_weight, W1, b1, W2, b2):
    src = edge_index[0].astype(jnp.int32)
    dst = edge_index[1].astype(jnp.int32)
    ew = edge_weight.astype(jnp.float32)
    npad_e = EPAD - E
    src = jnp.concatenate([src, jnp.zeros((npad_e,), jnp.int32)])
    dst = jnp.concatenate([dst, jnp.zeros((npad_e,), jnp.int32)])
    ew = jnp.concatenate([ew, jnp.zeros((npad_e,), jnp.float32)])

    deg0, deg1 = _deg_call(dst, ew)
    xw = _tcmm(x, W1)
    d0 = deg0.reshape(NPAD, 1)
    d1 = deg1.reshape(NPAD, 1)

    y1, dinv = _tc1(xw, d0, d1)

    src3 = src.reshape(NCH_TOT, CH)
    dst3 = dst.reshape(NCH_TOT, CH)
    ew3 = ew.reshape(NCH_TOT, CH)

    p1 = _agg_h(y1, src3, dst3, ew3)

    y2 = _tc2(p1, y1, dinv, b1.reshape(1, D_H), W2)

    p2 = _agg_o(y2, src3, dst3, ew3)

    return _tc3(p2, y2, dinv, b2.reshape(1, D_OUT))
